# Initial kernel scaffold; baseline (speedup 1.0000x reference)
import jax
import jax.numpy as jnp
from jax.experimental import pallas as pl


def kernel(points, verts, faces):
    raise NotImplementedError("write your pallas kernel here")


# speedup vs baseline: 4.1545x; 4.1545x over previous
import functools

import jax
import jax.numpy as jnp
from jax.experimental import pallas as pl
from jax.experimental.pallas import tpu as pltpu

ALPHA = 100.0
_EPS = 1e-12


def _rast_kernel(points_ref, verts_ref, faces_t_ref, out_ref, *, n_chunk):
    N = points_ref.shape[1]
    V = verts_ref.shape[1]
    F = faces_t_ref.shape[2]

    vx = verts_ref[0][:, 0:1]
    vy = verts_ref[0][:, 1:2]
    vz = verts_ref[0][:, 2:3]
    iota_v = jax.lax.broadcasted_iota(jnp.int32, (V, F), 0)

    tri = []
    for k in range(3):
        fk = faces_t_ref[0, k : k + 1, :]
        oh = iota_v == fk
        tri.append(
            tuple(
                jnp.sum(jnp.where(oh, vcoord, 0.0), axis=0, keepdims=True)
                for vcoord in (vx, vy, vz)
            )
        )
    (ax, ay, az), (bx, by, bz), (cx, cy, cz) = tri

    abx, aby, abz = bx - ax, by - ay, bz - az
    acx, acy, acz = cx - ax, cy - ay, cz - az
    bcx, bcy, bcz = cx - bx, cy - by, cz - bz

    ab2 = abx * abx + aby * aby + abz * abz
    ac2 = acx * acx + acy * acy + acz * acz
    bc2 = bcx * bcx + bcy * bcy + bcz * bcz
    dabac = abx * acx + aby * acy + abz * acz

    iab2 = 1.0 / jnp.maximum(ab2, _EPS)
    iac2 = 1.0 / jnp.maximum(ac2, _EPS)
    ibc2 = 1.0 / jnp.maximum(bc2, _EPS)

    nx = aby * acz - abz * acy
    ny = abz * acx - abx * acz
    nz = abx * acy - aby * acx
    n2 = nx * nx + ny * ny + nz * nz
    in2 = 1.0 / jnp.maximum(n2, _EPS)

    def seg_d2(p2, d, l2, il2):
        t = jnp.clip(d * il2, 0.0, 1.0)
        return p2 - t * (d + d - t * l2)

    for i in range(N // n_chunk):
        pts = points_ref[0, pl.ds(i * n_chunk, n_chunk), :]
        px = pts[:, 0:1]
        py = pts[:, 1:2]
        pz = pts[:, 2:3]

        apx = px - ax
        apy = py - ay
        apz = pz - az
        d1 = apx * abx + apy * aby + apz * abz
        d2 = apx * acx + apy * acy + apz * acz
        ap2 = apx * apx + apy * apy + apz * apz

        bpx = apx - abx
        bpy = apy - aby
        bpz = apz - abz
        bp2 = bpx * bpx + bpy * bpy + bpz * bpz

        d3 = d1 - ab2
        d4 = d2 - dabac
        d5 = d1 - dabac
        d6 = d2 - ac2
        dbc_b = d4 - d3
        dbc_c = d5 - d6

        vc = d1 * d4 - d3 * d2
        vb = d5 * d2 - d1 * d6
        va = d3 * d6 - d5 * d4

        e_ab = seg_d2(ap2, d1, ab2, iab2)
        e_ac = seg_d2(ap2, d2, ac2, iac2)
        e_bc = seg_d2(bp2, dbc_b, bc2, ibc2)

        dn = apx * nx + apy * ny + apz * nz
        pd = dn * dn * in2

        c1 = (d1 <= 0.0) & (d2 <= 0.0)
        c2 = (d3 >= 0.0) & (d4 <= d3)
        c3 = (d6 >= 0.0) & (d5 <= d6)
        c4 = (vc <= 0.0) & (d1 >= 0.0) & (d3 <= 0.0)
        c5 = (vb <= 0.0) & (d2 >= 0.0) & (d6 <= 0.0)
        c6 = (va <= 0.0) & (dbc_b >= 0.0) & (dbc_c >= 0.0)

        r = pd
        r = jnp.where(c6, e_bc, r)
        r = jnp.where(c5, e_ac, r)
        r = jnp.where(c4, e_ab, r)
        r = jnp.where(c3, e_ac, r)
        r = jnp.where(c2, e_ab, r)
        r = jnp.where(c1, e_ab, r)

        row_min = jnp.maximum(jnp.min(r, axis=1, keepdims=True), 0.0)
        out_ref[0, pl.ds(i * n_chunk, n_chunk), :] = jnp.exp(-ALPHA * row_min)


def kernel(points, verts, faces):
    B, N, _ = points.shape
    V = verts.shape[1]
    F = faces.shape[1]
    faces_t = jnp.transpose(faces, (0, 2, 1))

    out = pl.pallas_call(
        functools.partial(_rast_kernel, n_chunk=512),
        grid=(B,),
        in_specs=[
            pl.BlockSpec((1, N, 3), lambda i: (i, 0, 0)),
            pl.BlockSpec((1, V, 3), lambda i: (i, 0, 0)),
            pl.BlockSpec((1, 3, F), lambda i: (i, 0, 0)),
        ],
        out_specs=pl.BlockSpec((1, N, 1), lambda i: (i, 0, 0)),
        out_shape=jax.ShapeDtypeStruct((B, N, 1), jnp.float32),
    )(points, verts, faces_t)
    return out.reshape(B, N)


# speedup vs baseline: 5.5293x; 1.3309x over previous
import functools

import jax
import jax.numpy as jnp
from jax.experimental import pallas as pl
from jax.experimental.pallas import tpu as pltpu

ALPHA = 100.0
_EPS = 1e-12


def _rast_kernel(points_ref, verts_ref, faces_t_ref, out_ref, *, n_chunk):
    N = points_ref.shape[1]
    V = verts_ref.shape[1]
    F = faces_t_ref.shape[2]

    vx = verts_ref[0][:, 0:1]
    vy = verts_ref[0][:, 1:2]
    vz = verts_ref[0][:, 2:3]
    iota_v = jax.lax.broadcasted_iota(jnp.int32, (V, F), 0)

    tri = []
    for k in range(3):
        fk = faces_t_ref[0, k : k + 1, :]
        oh = iota_v == fk
        tri.append(
            tuple(
                jnp.sum(jnp.where(oh, vcoord, 0.0), axis=0, keepdims=True)
                for vcoord in (vx, vy, vz)
            )
        )
    (ax, ay, az), (bx, by, bz), (cx, cy, cz) = tri

    abx, aby, abz = bx - ax, by - ay, bz - az
    acx, acy, acz = cx - ax, cy - ay, cz - az
    bcx, bcy, bcz = cx - bx, cy - by, cz - bz

    ab2 = abx * abx + aby * aby + abz * abz
    ac2 = acx * acx + acy * acy + acz * acz
    bc2 = bcx * bcx + bcy * bcy + bcz * bcz
    dabac = abx * acx + aby * acy + abz * acz

    iab2 = 1.0 / jnp.maximum(ab2, _EPS)
    iac2 = 1.0 / jnp.maximum(ac2, _EPS)
    ibc2 = 1.0 / jnp.maximum(bc2, _EPS)

    nx = aby * acz - abz * acy
    ny = abz * acx - abx * acz
    nz = abx * acy - aby * acx
    n2 = nx * nx + ny * ny + nz * nz
    in2 = 1.0 / jnp.maximum(n2, _EPS)

    rbc = ab2 - dabac
    m_ab0 = ab2 == 0.0
    m_seg = (ac2 == 0.0) | (bc2 == 0.0)

    for i in range(N // n_chunk):
        pts = points_ref[0, pl.ds(i * n_chunk, n_chunk), :]
        px = pts[:, 0:1]
        py = pts[:, 1:2]
        pz = pts[:, 2:3]

        apx = px - ax
        apy = py - ay
        apz = pz - az
        d1 = apx * abx + apy * aby + apz * abz
        d2 = apx * acx + apy * acy + apz * acz
        ap2 = apx * apx + apy * apy + apz * apz

        u1 = d1 + d1
        bp2 = (ap2 - u1) + ab2
        dbc_b = (d2 - d1) + rbc

        t1 = jnp.clip(d1 * iab2, 0.0, 1.0)
        e_ab = ap2 - t1 * (u1 - t1 * ab2)
        u2 = d2 + d2
        t2 = jnp.clip(d2 * iac2, 0.0, 1.0)
        e_ac = ap2 - t2 * (u2 - t2 * ac2)
        u3 = dbc_b + dbc_b
        t3 = jnp.clip(dbc_b * ibc2, 0.0, 1.0)
        e_bc = bp2 - t3 * (u3 - t3 * bc2)

        dn = apx * nx + apy * ny + apz * nz
        pd = dn * dn * in2

        vbn = ac2 * d1 - dabac * d2
        vcn = ab2 * d2 - dabac * d1
        inside = (vbn > 0.0) & (vcn > 0.0) & (vbn + vcn < n2)

        dmin = jnp.minimum(jnp.minimum(e_ab, e_ac), e_bc)
        r = jnp.where(inside, jnp.minimum(pd, dmin), dmin)
        r = jnp.where(m_seg, e_ab, r)
        d6 = d2 - ac2
        r = jnp.where(m_ab0, jnp.where(d6 >= 0.0, e_ac, ap2), r)

        row_min = jnp.maximum(jnp.min(r, axis=1, keepdims=True), 0.0)
        out_ref[0, pl.ds(i * n_chunk, n_chunk), :] = jnp.exp(-ALPHA * row_min)


def kernel(points, verts, faces):
    B, N, _ = points.shape
    V = verts.shape[1]
    F = faces.shape[1]
    faces_t = jnp.transpose(faces, (0, 2, 1))

    out = pl.pallas_call(
        functools.partial(_rast_kernel, n_chunk=512),
        grid=(B,),
        in_specs=[
            pl.BlockSpec((1, N, 3), lambda i: (i, 0, 0)),
            pl.BlockSpec((1, V, 3), lambda i: (i, 0, 0)),
            pl.BlockSpec((1, 3, F), lambda i: (i, 0, 0)),
        ],
        out_specs=pl.BlockSpec((1, N, 1), lambda i: (i, 0, 0)),
        out_shape=jax.ShapeDtypeStruct((B, N, 1), jnp.float32),
    )(points, verts, faces_t)
    return out.reshape(B, N)


# speedup vs baseline: 5.8408x; 1.0563x over previous
import functools

import jax
import jax.numpy as jnp
from jax.experimental import pallas as pl
from jax.experimental.pallas import tpu as pltpu

ALPHA = 100.0
_EPS = 1e-12


def _rast_kernel(points_ref, verts_ref, faces_t_ref, out_ref, *, n_chunk):
    N = points_ref.shape[1]
    V = verts_ref.shape[1]
    F = faces_t_ref.shape[2]

    vx = verts_ref[0][:, 0:1]
    vy = verts_ref[0][:, 1:2]
    vz = verts_ref[0][:, 2:3]
    iota_v = jax.lax.broadcasted_iota(jnp.int32, (V, F), 0)

    tri = []
    for k in range(3):
        fk = faces_t_ref[0, k : k + 1, :]
        oh = iota_v == fk
        tri.append(
            tuple(
                jnp.sum(jnp.where(oh, vcoord, 0.0), axis=0, keepdims=True)
                for vcoord in (vx, vy, vz)
            )
        )
    (ax, ay, az), (bx, by, bz), (cx, cy, cz) = tri

    abx, aby, abz = bx - ax, by - ay, bz - az
    acx, acy, acz = cx - ax, cy - ay, cz - az
    bcx, bcy, bcz = cx - bx, cy - by, cz - bz

    ab2 = abx * abx + aby * aby + abz * abz
    ac2 = acx * acx + acy * acy + acz * acz
    bc2 = bcx * bcx + bcy * bcy + bcz * bcz
    dabac = abx * acx + aby * acy + abz * acz

    iab2 = 1.0 / jnp.maximum(ab2, _EPS)
    iac2 = 1.0 / jnp.maximum(ac2, _EPS)
    ibc2 = 1.0 / jnp.maximum(bc2, _EPS)

    nx = aby * acz - abz * acy
    ny = abz * acx - abx * acz
    nz = abx * acy - aby * acx
    n2 = nx * nx + ny * ny + nz * nz
    in2 = 1.0 / jnp.maximum(n2, _EPS)

    rbc = ab2 - dabac
    m_ab0 = ab2 == 0.0

    for i in range(N // n_chunk):
        pts = points_ref[0, pl.ds(i * n_chunk, n_chunk), :]
        px = pts[:, 0:1]
        py = pts[:, 1:2]
        pz = pts[:, 2:3]

        apx = px - ax
        apy = py - ay
        apz = pz - az
        d1 = apx * abx + apy * aby + apz * abz
        d2 = apx * acx + apy * acy + apz * acz
        ap2 = apx * apx + apy * apy + apz * apz

        u1 = d1 + d1
        bp2 = (ap2 - u1) + ab2
        dbc_b = (d2 - d1) + rbc

        t1 = jnp.clip(d1 * iab2, 0.0, 1.0)
        e_ab = ap2 - t1 * (u1 - t1 * ab2)
        u2 = d2 + d2
        t2 = jnp.clip(d2 * iac2, 0.0, 1.0)
        e_ac = ap2 - t2 * (u2 - t2 * ac2)
        u3 = dbc_b + dbc_b
        t3 = jnp.clip(dbc_b * ibc2, 0.0, 1.0)
        e_bc = bp2 - t3 * (u3 - t3 * bc2)

        vbn = ac2 * d1 - dabac * d2
        vcn = ab2 * d2 - dabac * d1
        s = vbn + vcn
        ins = jnp.minimum(jnp.minimum(vbn, vcn), n2 - s)
        pd = ap2 - (vbn * d1 + vcn * d2) * in2

        dmin = jnp.minimum(jnp.minimum(e_ab, e_ac), e_bc)
        r = jnp.minimum(jnp.where(ins > 0.0, pd, dmin), dmin)
        d6 = d2 - ac2
        r = jnp.where(m_ab0, jnp.where(d6 >= 0.0, e_ac, ap2), r)

        row_min = jnp.maximum(jnp.min(r, axis=1, keepdims=True), 0.0)
        out_ref[0, pl.ds(i * n_chunk, n_chunk), :] = jnp.exp(-ALPHA * row_min)


def kernel(points, verts, faces):
    B, N, _ = points.shape
    V = verts.shape[1]
    F = faces.shape[1]
    faces_t = jnp.transpose(faces, (0, 2, 1))

    out = pl.pallas_call(
        functools.partial(_rast_kernel, n_chunk=512),
        grid=(B,),
        in_specs=[
            pl.BlockSpec((1, N, 3), lambda i: (i, 0, 0)),
            pl.BlockSpec((1, V, 3), lambda i: (i, 0, 0)),
            pl.BlockSpec((1, 3, F), lambda i: (i, 0, 0)),
        ],
        out_specs=pl.BlockSpec((1, N, 1), lambda i: (i, 0, 0)),
        out_shape=jax.ShapeDtypeStruct((B, N, 1), jnp.float32),
    )(points, verts, faces_t)
    return out.reshape(B, N)
